# baseline (speedup 1.0000x reference)
import functools

import jax
import jax.numpy as jnp
from jax import lax
from jax.experimental import pallas as pl
from jax.experimental.pallas import tpu as pltpu
from jax.experimental.pallas import tpu_sc as plsc

_NC = 2
_NS = 16
_NW = _NC * _NS
_NRG = 8
_NCG = 4
_LANES = 16
_CHUNK = 112


def _rowgroup_bounds_py(rg, n):
    lo = ((rg * n) // _NRG) // 8 * 8
    hi = (((rg + 1) * n) // _NRG) // 8 * 8 if rg + 1 < _NRG else n
    return lo, hi


@functools.partial(jax.jit, static_argnums=(2, 3, 4))
def _sc_partial_segsum(feat, ids, n, d, nchunk):
    s_out = 512
    dummy = s_out
    acc_rows = s_out + 8
    dq = d // _NCG
    ncb = dq // _LANES
    niter = -(-nchunk // 2)

    mesh = plsc.VectorSubcoreMesh(core_axis_name="c", subcore_axis_name="s")

    @functools.partial(
        pl.kernel,
        out_type=jax.ShapeDtypeStruct((_NRG * s_out, d), jnp.float32),
        mesh=mesh,
        scratch_types=[
            pltpu.VMEM((_CHUNK,), jnp.int32),
            pltpu.VMEM((_CHUNK,), jnp.int32),
            pltpu.VMEM((_CHUNK, dq), jnp.float32),
            pltpu.VMEM((_CHUNK, dq), jnp.float32),
            pltpu.VMEM((acc_rows, dq), jnp.float32),
            pltpu.VMEM_SHARED((2 * (_NRG // _NC) * _CHUNK, d), jnp.float32),
            pltpu.SemaphoreType.DMA,
            pltpu.SemaphoreType.DMA,
            pltpu.SemaphoreType.DMA,
            pltpu.SemaphoreType.DMA,
        ],
    )
    def k(feat_hbm, ids_hbm, part_hbm, idx0, idx1, rows0, rows1, acc_v,
          stage_sh, semi0, semi1, semh0, semh1):
        cid = lax.axis_index("c")
        sid = lax.axis_index("s")
        wid = cid * _NS + sid
        rg = wid // _NCG
        rgl = rg % (_NRG // _NC)
        q = wid % _NCG

        lo = (rg * n) // _NRG // 8 * 8
        hi_raw = ((rg + 1) * n) // _NRG // 8 * 8
        hi = jnp.where(rg == _NRG - 1, n, hi_raw)
        iota = jnp.arange(_LANES, dtype=jnp.int32)

        idx_b = (idx0, idx1)
        rows_b = (rows0, rows1)
        semi_b = (semi0, semi1)
        semh_b = (semh0, semh1)
        nquad = _NRG // _NC

        def a_of(kk):
            return jnp.minimum(lo + kk * _CHUNK, hi - _CHUNK)

        def slab(b):
            return pl.ds((b * nquad + rgl) * _CHUNK, _CHUNK)

        def hbm_start(b, kk):
            @pl.when(q == 0)
            def _():
                pltpu.async_copy(
                    feat_hbm.at[pl.ds(a_of(kk), _CHUNK)],
                    stage_sh.at[slab(b)],
                    semh_b[b],
                )

        def hbm_wait(b):
            @pl.when(q == 0)
            def _():
                pltpu.make_async_copy(
                    feat_hbm.at[pl.ds(0, _CHUNK)], stage_sh.at[slab(b)], semh_b[b]
                ).wait()

        def ids_start(b, kk):
            pltpu.async_copy(
                ids_hbm.at[pl.ds(a_of(kk), _CHUNK)], idx_b[b], semi_b[b]
            )

        def ids_wait(b):
            pltpu.make_async_copy(
                ids_hbm.at[pl.ds(0, _CHUNK)], idx_b[b], semi_b[b]
            ).wait()

        def stage_fetch(b):
            pltpu.sync_copy(
                stage_sh.at[slab(b), pl.ds(q * dq, dq)], rows_b[b]
            )

        hbm_start(0, 0)
        hbm_start(1, 1)
        ids_start(0, 0)
        ids_start(1, 1)
        zeros16 = jnp.zeros((_LANES,), jnp.float32)

        def _zero_row(r, _):
            for cb in range(ncb):
                acc_v[r, pl.ds(cb * _LANES, _LANES)] = zeros16
            return 0

        lax.fori_loop(0, acc_rows, _zero_row, 0)

        def process(b, kk):
            s_k = lo + kk * _CHUNK
            a_k = a_of(kk)
            idx_v = idx_b[b]
            rows_v = rows_b[b]
            @plsc.parallel_loop(0, _CHUNK // _LANES)
            def _group(g):
                glb = a_k + g * _LANES + iota
                raw = idx_v[pl.ds(g * _LANES, _LANES)]
                idv = jnp.where(glb >= s_k, raw, jnp.int32(dummy))
                sidx0 = idv[0]
                uniform = sidx0 == idv[_LANES - 1]

                @pl.when(uniform)
                def _fast():
                    def _tree_store(cb, vs):
                        while len(vs) > 1:
                            vs = [
                                vs[i] + vs[i + 1] if i + 1 < len(vs) else vs[i]
                                for i in range(0, len(vs), 2)
                            ]
                        plsc.addupdate(
                            acc_v.at[sidx0, pl.ds(cb * _LANES, _LANES)], vs[0]
                        )

                    prev = None
                    for cb in range(ncb):
                        csl = pl.ds(cb * _LANES, _LANES)
                        cur = [rows_v[g * _LANES + l, csl] for l in range(_LANES)]
                        if prev is not None:
                            _tree_store(prev[0], prev[1])
                        prev = (cb, cur)
                    _tree_store(prev[0], prev[1])

                @pl.when(jnp.logical_not(uniform))
                def _slow():
                    sidxs = [idv[l] for l in range(_LANES)]
                    csls = [pl.ds(cb * _LANES, _LANES) for cb in range(ncb)]

                    def _stores(l, vals):
                        for cb in range(ncb):
                            plsc.addupdate(acc_v.at[sidxs[l], csls[cb]], vals[cb])

                    prev = None
                    for l in range(_LANES):
                        cur = [rows_v[g * _LANES + l, csls[cb]] for cb in range(ncb)]
                        if prev is not None:
                            _stores(prev[0], prev[1])
                        prev = (l, cur)
                    _stores(prev[0], prev[1])


        def _half(b, kk):
            hbm_wait(b)
            plsc.subcore_barrier()
            stage_fetch(b)
            plsc.subcore_barrier()
            hbm_start(b, kk + 2)
            ids_wait(b)
            process(b, kk)
            ids_start(b, kk + 2)

        def _iter(i, _):
            kk0 = 2 * i
            _half(0, kk0)
            _half(1, kk0 + 1)
            return 0

        lax.fori_loop(0, niter, _iter, 0)
        hbm_wait(0)
        hbm_wait(1)
        ids_wait(0)
        ids_wait(1)

        pltpu.sync_copy(
            acc_v.at[pl.ds(0, s_out)],
            part_hbm.at[pl.ds(rg * s_out, s_out), pl.ds(q * dq, dq)],
        )

    return k(feat, ids)


def _combine_body(p_ref, o_ref):
    acc = p_ref[0]
    for r in range(1, _NRG):
        acc = acc + p_ref[r]
    o_ref[...] = acc


def kernel(feat, segment_ids, num_segments):
    n, d = feat.shape
    assert n % 8 == 0 and d % (_NCG * _LANES) == 0
    ids = jnp.minimum(
        segment_ids, jnp.asarray(num_segments, segment_ids.dtype) - 1
    ).astype(jnp.int32)

    bounds = [_rowgroup_bounds_py(rg, n) for rg in range(_NRG)]
    rows = [hi - lo for lo, hi in bounds]
    nchunk = -(-max(rows) // _CHUNK)
    assert min(rows) >= _CHUNK and (nchunk - 1) * _CHUNK < min(rows)

    partial = _sc_partial_segsum(feat, ids, n, d, nchunk)
    p2 = partial.reshape(_NRG, 512, d)
    out = pl.pallas_call(
        _combine_body,
        out_shape=jax.ShapeDtypeStruct((512, d), jnp.float32),
    )(p2)
    return out


# speedup vs baseline: 1.3908x; 1.3908x over previous
import functools

import jax
import jax.numpy as jnp
from jax import lax
from jax.experimental import pallas as pl
from jax.experimental.pallas import tpu as pltpu
from jax.experimental.pallas import tpu_sc as plsc

_NC = 2
_NS = 16
_NW = _NC * _NS
_NRG = 8
_NCG = 4
_LANES = 16
_CHUNK = 192


def _rowgroup_bounds_py(rg, n):
    lo = ((rg * n) // _NRG) // 8 * 8
    hi = (((rg + 1) * n) // _NRG) // 8 * 8 if rg + 1 < _NRG else n
    return lo, hi


@functools.partial(jax.jit, static_argnums=(2, 3, 4))
def _sc_partial_segsum(feat, ids, n, d, nchunk):
    s_out = 512
    dummy = s_out
    acc_rows = s_out + 8
    dq = d // _NCG
    ncb = dq // _LANES
    niter = -(-nchunk // 2)

    mesh = plsc.VectorSubcoreMesh(core_axis_name="c", subcore_axis_name="s")

    @functools.partial(
        pl.kernel,
        out_type=jax.ShapeDtypeStruct((_NRG * s_out, d), jnp.float32),
        mesh=mesh,
        scratch_types=[
            pltpu.VMEM((_CHUNK,), jnp.int32),
            pltpu.VMEM((_CHUNK,), jnp.int32),
            pltpu.VMEM((_CHUNK, dq), jnp.float32),
            pltpu.VMEM((_CHUNK, dq), jnp.float32),
            pltpu.VMEM((acc_rows, dq), jnp.float32),
            pltpu.SemaphoreType.DMA,
            pltpu.SemaphoreType.DMA,
            pltpu.SemaphoreType.DMA,
            pltpu.SemaphoreType.DMA,
        ],
    )
    def k(feat_hbm, ids_hbm, part_hbm, idx0, idx1, rows0, rows1, acc_v,
          semi0, semi1, semf0, semf1):
        cid = lax.axis_index("c")
        sid = lax.axis_index("s")
        wid = sid * _NC + cid
        rg = wid // _NCG
        q = wid % _NCG

        lo = (rg * n) // _NRG // 8 * 8
        hi_raw = ((rg + 1) * n) // _NRG // 8 * 8
        hi = jnp.where(rg == _NRG - 1, n, hi_raw)
        iota = jnp.arange(_LANES, dtype=jnp.int32)

        idx_b = (idx0, idx1)
        rows_b = (rows0, rows1)
        semi_b = (semi0, semi1)
        semf_b = (semf0, semf1)

        def a_of(kk):
            return jnp.minimum(lo + kk * _CHUNK, hi - _CHUNK)

        def start(b, kk):
            a_k = a_of(kk)
            pltpu.async_copy(ids_hbm.at[pl.ds(a_k, _CHUNK)], idx_b[b], semi_b[b])
            pltpu.async_copy(
                feat_hbm.at[pl.ds(a_k, _CHUNK), pl.ds(q * dq, dq)],
                rows_b[b],
                semf_b[b],
            )

        def wait(b):
            pltpu.make_async_copy(
                ids_hbm.at[pl.ds(0, _CHUNK)], idx_b[b], semi_b[b]
            ).wait()
            pltpu.make_async_copy(
                feat_hbm.at[pl.ds(0, _CHUNK), pl.ds(0, dq)], rows_b[b], semf_b[b]
            ).wait()

        start(0, 0)
        start(1, 1)
        zeros16 = jnp.zeros((_LANES,), jnp.float32)

        def _zero_row(r, _):
            for cb in range(ncb):
                acc_v[r, pl.ds(cb * _LANES, _LANES)] = zeros16
            return 0

        lax.fori_loop(0, acc_rows, _zero_row, 0)

        def process(b, kk):
            s_k = lo + kk * _CHUNK
            a_k = a_of(kk)
            idx_v = idx_b[b]
            rows_v = rows_b[b]
            @plsc.parallel_loop(0, _CHUNK // _LANES)
            def _group(g):
                glb = a_k + g * _LANES + iota
                raw = idx_v[pl.ds(g * _LANES, _LANES)]
                idv = jnp.where(glb >= s_k, raw, jnp.int32(dummy))
                sidx0 = idv[0]
                uniform = sidx0 == idv[_LANES - 1]

                @pl.when(uniform)
                def _fast():
                    def _tree_store(cb, vs):
                        while len(vs) > 1:
                            vs = [
                                vs[i] + vs[i + 1] if i + 1 < len(vs) else vs[i]
                                for i in range(0, len(vs), 2)
                            ]
                        plsc.addupdate(
                            acc_v.at[sidx0, pl.ds(cb * _LANES, _LANES)], vs[0]
                        )

                    prev = None
                    for cb in range(ncb):
                        csl = pl.ds(cb * _LANES, _LANES)
                        cur = [rows_v[g * _LANES + l, csl] for l in range(_LANES)]
                        if prev is not None:
                            _tree_store(prev[0], prev[1])
                        prev = (cb, cur)
                    _tree_store(prev[0], prev[1])

                @pl.when(jnp.logical_not(uniform))
                def _slow():
                    sidxs = [idv[l] for l in range(_LANES)]
                    csls = [pl.ds(cb * _LANES, _LANES) for cb in range(ncb)]

                    def _stores(l, vals):
                        for cb in range(ncb):
                            plsc.addupdate(acc_v.at[sidxs[l], csls[cb]], vals[cb])

                    prev = None
                    for l in range(_LANES):
                        cur = [rows_v[g * _LANES + l, csls[cb]] for cb in range(ncb)]
                        if prev is not None:
                            _stores(prev[0], prev[1])
                        prev = (l, cur)
                    _stores(prev[0], prev[1])


        def _iter(i, _):
            kk0 = 2 * i
            wait(0)
            process(0, kk0)
            start(0, kk0 + 2)
            wait(1)
            process(1, kk0 + 1)
            start(1, kk0 + 3)
            return 0

        lax.fori_loop(0, niter, _iter, 0)
        wait(0)
        wait(1)

        pltpu.sync_copy(
            acc_v.at[pl.ds(0, s_out)],
            part_hbm.at[pl.ds(rg * s_out, s_out), pl.ds(q * dq, dq)],
        )

    return k(feat, ids)


def _combine_body(p_ref, o_ref):
    acc = p_ref[0]
    for r in range(1, _NRG):
        acc = acc + p_ref[r]
    o_ref[...] = acc


def kernel(feat, segment_ids, num_segments):
    n, d = feat.shape
    assert n % 8 == 0 and d % (_NCG * _LANES) == 0
    ids = jnp.minimum(
        segment_ids, jnp.asarray(num_segments, segment_ids.dtype) - 1
    ).astype(jnp.int32)

    bounds = [_rowgroup_bounds_py(rg, n) for rg in range(_NRG)]
    rows = [hi - lo for lo, hi in bounds]
    nchunk = -(-max(rows) // _CHUNK)
    assert min(rows) >= _CHUNK and (nchunk - 1) * _CHUNK < min(rows)

    partial = _sc_partial_segsum(feat, ids, n, d, nchunk)
    p2 = partial.reshape(_NRG, 512, d)
    out = pl.pallas_call(
        _combine_body,
        out_shape=jax.ShapeDtypeStruct((512, d), jnp.float32),
    )(p2)
    return out


# speedup vs baseline: 1.4353x; 1.0320x over previous
import functools

import jax
import jax.numpy as jnp
from jax import lax
from jax.experimental import pallas as pl
from jax.experimental.pallas import tpu as pltpu
from jax.experimental.pallas import tpu_sc as plsc

_NC = 2
_NS = 16
_NW = _NC * _NS
_NRG = 8
_NCG = 4
_LANES = 16
_CHUNK = 224


def _rowgroup_bounds_py(rg, n):
    lo = ((rg * n) // _NRG) // 8 * 8
    hi = (((rg + 1) * n) // _NRG) // 8 * 8 if rg + 1 < _NRG else n
    return lo, hi


@functools.partial(jax.jit, static_argnums=(2, 3, 4))
def _sc_partial_segsum(feat, ids, n, d, nchunk):
    s_out = 512
    dummy = s_out
    acc_rows = s_out + 8
    dq = d // _NCG
    ncb = dq // _LANES
    niter = -(-nchunk // 2)

    mesh = plsc.VectorSubcoreMesh(core_axis_name="c", subcore_axis_name="s")

    @functools.partial(
        pl.kernel,
        out_type=jax.ShapeDtypeStruct((_NRG * s_out, d), jnp.float32),
        mesh=mesh,
        scratch_types=[
            pltpu.VMEM((_CHUNK,), jnp.int32),
            pltpu.VMEM((_CHUNK,), jnp.int32),
            pltpu.VMEM((_CHUNK, dq), jnp.float32),
            pltpu.VMEM((_CHUNK, dq), jnp.float32),
            pltpu.VMEM((acc_rows, dq), jnp.float32),
            pltpu.SemaphoreType.DMA,
            pltpu.SemaphoreType.DMA,
            pltpu.SemaphoreType.DMA,
            pltpu.SemaphoreType.DMA,
        ],
    )
    def k(feat_hbm, ids_hbm, part_hbm, idx0, idx1, rows0, rows1, acc_v,
          semi0, semi1, semf0, semf1):
        cid = lax.axis_index("c")
        sid = lax.axis_index("s")
        wid = sid * _NC + cid
        rg = wid // _NCG
        q = wid % _NCG

        lo = (rg * n) // _NRG // 8 * 8
        hi_raw = ((rg + 1) * n) // _NRG // 8 * 8
        hi = jnp.where(rg == _NRG - 1, n, hi_raw)
        iota = jnp.arange(_LANES, dtype=jnp.int32)

        idx_b = (idx0, idx1)
        rows_b = (rows0, rows1)
        semi_b = (semi0, semi1)
        semf_b = (semf0, semf1)

        def a_of(kk):
            return jnp.minimum(lo + kk * _CHUNK, hi - _CHUNK)

        def start(b, kk):
            a_k = a_of(kk)
            pltpu.async_copy(ids_hbm.at[pl.ds(a_k, _CHUNK)], idx_b[b], semi_b[b])
            pltpu.async_copy(
                feat_hbm.at[pl.ds(a_k, _CHUNK), pl.ds(q * dq, dq)],
                rows_b[b],
                semf_b[b],
            )

        def wait(b):
            pltpu.make_async_copy(
                ids_hbm.at[pl.ds(0, _CHUNK)], idx_b[b], semi_b[b]
            ).wait()
            pltpu.make_async_copy(
                feat_hbm.at[pl.ds(0, _CHUNK), pl.ds(0, dq)], rows_b[b], semf_b[b]
            ).wait()

        start(0, 0)
        start(1, 1)
        zeros16 = jnp.zeros((_LANES,), jnp.float32)

        def _zero_row(r, _):
            for cb in range(ncb):
                acc_v[r, pl.ds(cb * _LANES, _LANES)] = zeros16
            return 0

        lax.fori_loop(0, acc_rows, _zero_row, 0)

        def process(b, kk):
            s_k = lo + kk * _CHUNK
            a_k = a_of(kk)
            idx_v = idx_b[b]
            rows_v = rows_b[b]
            @plsc.parallel_loop(0, _CHUNK // _LANES)
            def _group(g):
                glb = a_k + g * _LANES + iota
                raw = idx_v[pl.ds(g * _LANES, _LANES)]
                idv = jnp.where(glb >= s_k, raw, jnp.int32(dummy))
                sidx0 = idv[0]
                uniform = sidx0 == idv[_LANES - 1]

                @pl.when(uniform)
                def _fast():
                    def _tree_store(cb, vs):
                        while len(vs) > 1:
                            vs = [
                                vs[i] + vs[i + 1] if i + 1 < len(vs) else vs[i]
                                for i in range(0, len(vs), 2)
                            ]
                        plsc.addupdate(
                            acc_v.at[sidx0, pl.ds(cb * _LANES, _LANES)], vs[0]
                        )

                    prev = None
                    for cb in range(ncb):
                        csl = pl.ds(cb * _LANES, _LANES)
                        cur = [rows_v[g * _LANES + l, csl] for l in range(_LANES)]
                        if prev is not None:
                            _tree_store(prev[0], prev[1])
                        prev = (cb, cur)
                    _tree_store(prev[0], prev[1])

                @pl.when(jnp.logical_not(uniform))
                def _slow():
                    sidxs = [idv[l] for l in range(_LANES)]
                    csls = [pl.ds(cb * _LANES, _LANES) for cb in range(ncb)]

                    def _stores(l, vals):
                        for cb in range(ncb):
                            plsc.addupdate(acc_v.at[sidxs[l], csls[cb]], vals[cb])

                    prev = None
                    for l in range(_LANES):
                        cur = [rows_v[g * _LANES + l, csls[cb]] for cb in range(ncb)]
                        if prev is not None:
                            _stores(prev[0], prev[1])
                        prev = (l, cur)
                    _stores(prev[0], prev[1])


        def _iter(i, _):
            kk0 = 2 * i
            wait(0)
            process(0, kk0)
            start(0, kk0 + 2)
            wait(1)
            process(1, kk0 + 1)
            start(1, kk0 + 3)
            return 0

        lax.fori_loop(0, niter, _iter, 0)
        wait(0)
        wait(1)

        pltpu.sync_copy(
            acc_v.at[pl.ds(0, s_out)],
            part_hbm.at[pl.ds(rg * s_out, s_out), pl.ds(q * dq, dq)],
        )

    return k(feat, ids)


def _combine_body(p_ref, o_ref):
    acc = p_ref[0]
    for r in range(1, _NRG):
        acc = acc + p_ref[r]
    o_ref[...] = acc


def kernel(feat, segment_ids, num_segments):
    n, d = feat.shape
    assert n % 8 == 0 and d % (_NCG * _LANES) == 0
    ids = jnp.minimum(
        segment_ids, jnp.asarray(num_segments, segment_ids.dtype) - 1
    ).astype(jnp.int32)

    bounds = [_rowgroup_bounds_py(rg, n) for rg in range(_NRG)]
    rows = [hi - lo for lo, hi in bounds]
    nchunk = -(-max(rows) // _CHUNK)
    assert min(rows) >= _CHUNK and (nchunk - 1) * _CHUNK < min(rows)

    partial = _sc_partial_segsum(feat, ids, n, d, nchunk)
    p2 = partial.reshape(_NRG, 512, d)
    out = pl.pallas_call(
        _combine_body,
        out_shape=jax.ShapeDtypeStruct((512, d), jnp.float32),
    )(p2)
    return out
